# Initial kernel scaffold; baseline (speedup 1.0000x reference)
import jax
import jax.numpy as jnp
from jax.experimental import pallas as pl


def kernel(x, table):
    raise NotImplementedError("write your pallas kernel here")


# speedup vs baseline: 2.8788x; 2.8788x over previous
import functools

import jax
import jax.numpy as jnp
from jax import lax
from jax.experimental import pallas as pl
from jax.experimental.pallas import tpu as pltpu
from jax.experimental.pallas import tpu_sc as plsc

BATCH = 4
SEQ = 4096
DIM = 1024

_NUM_CORES = 2
_NUM_SUBCORES = 16
_NW = _NUM_CORES * _NUM_SUBCORES
_ROWS_PER_W = SEQ // _NW
_CHUNK = 32
_NCHUNK = _ROWS_PER_W // _CHUNK


def _run(table):
    mesh = plsc.VectorSubcoreMesh(core_axis_name="c", subcore_axis_name="s")

    @functools.partial(
        pl.kernel,
        mesh=mesh,
        out_type=jax.ShapeDtypeStruct((BATCH, SEQ, DIM), jnp.float32),
        scratch_types=[
            pltpu.VMEM((_CHUNK, DIM), jnp.float32),
        ],
    )
    def body(table_hbm, out_hbm, buf):
        wid = lax.axis_index("s") * _NUM_CORES + lax.axis_index("c")
        base = wid * _ROWS_PER_W
        for i in range(_NCHUNK):
            row0 = base + i * _CHUNK
            pltpu.sync_copy(table_hbm.at[pl.ds(row0, _CHUNK)], buf)
            for b in range(BATCH):
                pltpu.sync_copy(buf, out_hbm.at[b, pl.ds(row0, _CHUNK)])

    return body(table)


def kernel(x, table):
    del x
    return _run(table)


# speedup vs baseline: 2.9901x; 1.0387x over previous
import functools

import jax
import jax.numpy as jnp
from jax import lax
from jax.experimental import pallas as pl
from jax.experimental.pallas import tpu as pltpu
from jax.experimental.pallas import tpu_sc as plsc

BATCH = 4
SEQ = 4096
DIM = 1024

_NUM_CORES = 2
_NUM_SUBCORES = 16
_NW = _NUM_CORES * _NUM_SUBCORES
_ROWS_PER_W = SEQ // _NW
_CHUNK = 16
_NCHUNK = _ROWS_PER_W // _CHUNK
_NBUF = 7


def _run(table):
    mesh = plsc.VectorSubcoreMesh(core_axis_name="c", subcore_axis_name="s")

    @functools.partial(
        pl.kernel,
        mesh=mesh,
        out_type=jax.ShapeDtypeStruct((BATCH, SEQ, DIM), jnp.float32),
        scratch_types=(
            [pltpu.VMEM((_CHUNK, DIM), jnp.float32) for _ in range(_NBUF)]
            + [pltpu.SemaphoreType.DMA for _ in range(2 * _NBUF)]
        ),
    )
    def body(table_hbm, out_hbm, *scratch):
        bufs = scratch[:_NBUF]
        gsems = scratch[_NBUF : 2 * _NBUF]
        ssems = scratch[2 * _NBUF :]
        wid = lax.axis_index("s") * _NUM_CORES + lax.axis_index("c")
        base = wid * _ROWS_PER_W

        def gather(i):
            row0 = base + i * _CHUNK
            k = i % _NBUF
            return pltpu.async_copy(table_hbm.at[pl.ds(row0, _CHUNK)], bufs[k], gsems[k])

        def stores(i):
            row0 = base + i * _CHUNK
            k = i % _NBUF
            return [
                pltpu.async_copy(bufs[k], out_hbm.at[b, pl.ds(row0, _CHUNK)], ssems[k])
                for b in range(BATCH)
            ]

        g = {}
        st = {}
        waited = set()
        for i in range(min(_NBUF, _NCHUNK)):
            g[i] = gather(i)
        for i in range(_NCHUNK):
            g[i].wait()
            st[i] = stores(i)
            j = i + _NBUF
            if j < _NCHUNK:
                for h in st[i]:
                    h.wait()
                waited.add(i)
                g[j] = gather(j)
        for i in range(_NCHUNK):
            if i not in waited:
                for h in st[i]:
                    h.wait()

    return body(table)


def kernel(x, table):
    del x
    return _run(table)
